# baseline (device time: 47752 ns/iter reference)
import jax
import jax.numpy as jnp
from jax import lax
from jax.experimental import pallas as pl
from jax.experimental.pallas import tpu as pltpu

N_DEV = 32
B, SQ, DM = 2, 256, 768
HQ_SH, DH = 4, 64
DSH = HQ_SH * DH
ROWS = B * SQ
HALF = ROWS // 2
CHUNK = ROWS // N_DEV


def kernel(x, Wq, Wk, Wv, Wo):
    def body(x_ref, wq_ref, wk_ref, wv_ref, wo_ref, out_ref,
             part_ref, recv_ref, red_ref, send1, recv1, send2, recv2):
        my = lax.axis_index("i")
        mybatch = my // 16

        wq = wq_ref[...].astype(jnp.bfloat16)
        wk = wk_ref[...].astype(jnp.bfloat16)
        wv = wv_ref[...].astype(jnp.bfloat16)
        wo = wo_ref[...].astype(jnp.bfloat16)

        prow = lax.broadcasted_iota(jnp.int32, (SQ, DSH), 0).astype(jnp.float32)
        cols = lax.broadcasted_iota(jnp.int32, (SQ, DSH), 1)
        expo = (2 * ((cols % DH) // 2)).astype(jnp.float32) / DH
        ang = prow * jnp.exp(-expo * jnp.log(10000.0))
        cos_t = jnp.cos(ang)
        sin_t = jnp.sin(ang)
        even = (cols % 2) == 0

        def rot(t):
            zero = jnp.zeros((SQ, 1), t.dtype)
            tm1 = jnp.concatenate([t[:, 1:], zero], axis=1)
            tp1 = jnp.concatenate([zero, t[:, :-1]], axis=1)
            t_r = jnp.where(even, -tm1, tp1)
            return t * cos_t + t_r * sin_t

        def do_half(bidx):
            base = HALF * bidx
            xb = x_ref[pl.ds(bidx, 1), :, :].reshape(SQ, DM).astype(jnp.bfloat16)
            qb = rot(jnp.dot(xb, wq, preferred_element_type=jnp.float32))
            kb = rot(jnp.dot(xb, wk, preferred_element_type=jnp.float32))
            qb = qb.astype(jnp.bfloat16)
            kb = kb.astype(jnp.bfloat16)
            vb = jnp.dot(xb, wv,
                         preferred_element_type=jnp.float32).astype(jnp.bfloat16)
            ctx = []
            for h in range(HQ_SH):
                cb = slice(h * DH, (h + 1) * DH)
                qh, kh, vh = qb[:, cb], kb[:, cb], vb[:, cb]
                sc = lax.dot_general(
                    qh, kh, (((1,), (1,)), ((), ())),
                    preferred_element_type=jnp.float32) * 0.125
                m = jnp.max(sc, axis=-1, keepdims=True)
                e = jnp.exp(sc - m)
                w = (e / jnp.sum(e, axis=-1, keepdims=True)).astype(jnp.bfloat16)
                ctx.append(jnp.dot(w, vh, preferred_element_type=jnp.float32))
            ctx_b = jnp.concatenate(ctx, axis=1).astype(jnp.bfloat16)
            part_ref[pl.ds(base, HALF), :] = jnp.dot(
                ctx_b, wo, preferred_element_type=jnp.float32
            ).astype(jnp.bfloat16)

        def start_sends(ks):
            ds = []
            for k in ks:
                t = jnp.bitwise_xor(my, k)
                rd = pltpu.make_async_remote_copy(
                    src_ref=part_ref.at[pl.ds(CHUNK * t, CHUNK)],
                    dst_ref=recv_ref.at[k],
                    send_sem=send1.at[k],
                    recv_sem=recv1.at[k],
                    device_id=(t,),
                    device_id_type=pl.DeviceIdType.MESH,
                )
                rd.start()
                ds.append(rd)
            return ds

        do_half(mybatch)
        d1 = start_sends(range(1, 16))
        do_half(1 - mybatch)
        d1 += start_sends(range(16, 32))

        acc = part_ref[pl.ds(CHUNK * my, CHUNK), :].astype(jnp.float32)
        for k in range(1, N_DEV):
            d1[k - 1].wait_recv()
            acc = acc + recv_ref[k, :, :].astype(jnp.float32)
        red = acc.astype(jnp.bfloat16)
        out_ref[pl.ds(CHUNK * my, CHUNK), :] = red
        red_ref[...] = red

        d2 = []
        for k in range(1, N_DEV):
            rd = pltpu.make_async_remote_copy(
                src_ref=red_ref,
                dst_ref=out_ref.at[pl.ds(CHUNK * my, CHUNK)],
                send_sem=send2.at[k],
                recv_sem=recv2.at[k],
                device_id=(jnp.bitwise_xor(my, k),),
                device_id_type=pl.DeviceIdType.MESH,
            )
            rd.start()
            d2.append(rd)
        for rd in d2:
            rd.wait_recv()
        for rd in d1 + d2:
            rd.wait_send()

    out = pl.pallas_call(
        body,
        out_shape=jax.ShapeDtypeStruct((ROWS, DM), jnp.bfloat16),
        in_specs=[pl.BlockSpec(memory_space=pltpu.VMEM)] * 5,
        out_specs=pl.BlockSpec(memory_space=pltpu.VMEM),
        scratch_shapes=[
            pltpu.VMEM((ROWS, DM), jnp.bfloat16),
            pltpu.VMEM((N_DEV, CHUNK, DM), jnp.bfloat16),
            pltpu.VMEM((CHUNK, DM), jnp.bfloat16),
            pltpu.SemaphoreType.DMA((N_DEV,)),
            pltpu.SemaphoreType.DMA((N_DEV,)),
            pltpu.SemaphoreType.DMA((N_DEV,)),
            pltpu.SemaphoreType.DMA((N_DEV,)),
        ],
    )(x, Wq, Wk, Wv, Wo)
    return out.reshape(B, SQ, DM)
